# baseline (speedup 1.0000x reference)
import math

import numpy as np
import jax
import jax.numpy as jnp
from jax import lax
from jax.experimental import pallas as pl
from jax.experimental.pallas import tpu as pltpu
from jax.experimental.pallas import tpu_sc as plsc

ROWS, COLS = 64, 8192
N = ROWS * COLS
K = math.floor(0.05 * N)
C = 1024
NB = N // C
NT = 16
PT = N // NT
CW = C // NT
MANT = 0x7FFFFF


def _threefry2x32_np(k0, k1, x0, x1):
    def rotl(x, d):
        return ((x << np.uint32(d)) | (x >> np.uint32(32 - d))).astype(np.uint32)

    ks0 = np.uint32(k0)
    ks1 = np.uint32(k1)
    ks2 = np.uint32(ks0 ^ ks1 ^ np.uint32(0x1BD11BDA))
    x0 = (x0 + ks0).astype(np.uint32)
    x1 = (x1 + ks1).astype(np.uint32)
    rots = [[13, 15, 26, 6], [17, 29, 16, 24]]
    ks = [ks0, ks1, ks2]
    for i in range(5):
        for r in rots[i % 2]:
            x0 = (x0 + x1).astype(np.uint32)
            x1 = rotl(x1, r)
            x1 = (x1 ^ x0).astype(np.uint32)
        x0 = (x0 + ks[(i + 1) % 3]).astype(np.uint32)
        x1 = (x1 + ks[(i + 2) % 3] + np.uint32(i + 1)).astype(np.uint32)
    return x0, x1


def _score_mantissas_np(seed, n):
    counts = np.arange(n, dtype=np.uint64)
    o0, o1 = _threefry2x32_np(
        np.uint32(seed >> 32), np.uint32(seed & 0xFFFFFFFF),
        (counts >> np.uint64(32)).astype(np.uint32), counts.astype(np.uint32))
    return ((o0 ^ o1) >> np.uint32(9))


_u_np = _score_mantissas_np(42, N)
_perm_np = np.argsort(_u_np)[::-1].astype(np.int32)
_rank_np = np.empty(N, dtype=np.int32)
_rank_np[_perm_np] = np.arange(N, dtype=np.int32)
_seg_np = (_rank_np // C).astype(np.uint32)
_meta_np = ((_seg_np << np.uint32(23)) | _u_np).astype(np.uint32).view(np.int32)
_svali_np = _u_np[_perm_np].astype(np.uint32).view(np.int32)


def _sc_body(x_hbm, meta_hbm, sidx_hbm, svali_hbm, out_hbm,
             xv, metav, hist16, histg, histall, zflat, svalb,
             idxb, valb, zv, tvout, tvin, sema, semb, sh_hist, sh_z, sh_t):
    sid = lax.axis_index("s")
    zeros16 = jnp.zeros((16,), jnp.float32)
    iota_i = lax.iota(jnp.int32, 16)
    iota_f = iota_i.astype(jnp.float32)
    kf = jnp.float32(K)
    cf = jnp.float32(C)
    one_f = jnp.float32(1.0)
    zero_f = jnp.float32(0.0)

    base = sid * PT
    cpa = pltpu.async_copy(x_hbm.at[pl.ds(base, PT)], xv, sema)
    cpb = pltpu.async_copy(meta_hbm.at[pl.ds(base, PT)], metav, semb)

    def _zero_hist(i, c):
        for u in range(8):
            hist16[pl.ds(i * 128 + u * 16, 16)] = zeros16
        return c

    lax.fori_loop(0, (16 * NB) // 128, _zero_hist, 0)
    cpa.wait()
    cpb.wait()

    lane_off = iota_i * NB

    def _hist(i, c):
        for u in range(8):
            o = i * 128 + u * 16
            v = xv[pl.ds(o, 16)]
            m = metav[pl.ds(o, 16)]
            seg = lax.shift_right_logical(m, 23)
            ones = jnp.where(v == 0.0, one_f, zero_f)
            plsc.addupdate_scatter(hist16, [seg + lane_off], ones)
        return c

    lax.fori_loop(0, PT // 128, _hist, 0)

    def _lane_reduce(i, c):
        acc = zeros16
        for row in range(16):
            acc = acc + hist16[pl.ds(row * NB + i * 16, 16)]
        histg[pl.ds(i * 16, 16)] = acc
        return c

    lax.fori_loop(0, NB // 16, _lane_reduce, 0)
    pltpu.sync_copy(histg, sh_hist.at[pl.ds(sid * NB, NB)])
    plsc.subcore_barrier()

    pltpu.sync_copy(sh_hist, histall)

    def _combine(i, c):
        acc = zeros16
        for row in range(16):
            acc = acc + histall[pl.ds(row * NB + i * 16, 16)]
        histg[pl.ds(i * 16, 16)] = cf - acc
        return c

    lax.fori_loop(0, NB // 16, _combine, 0)

    def _select(i, carry):
        cum, bmin = carry
        nzc = histg[pl.ds(i * 16, 16)]
        cs = plsc.cumsum(nzc) + cum
        lane_g = iota_f + (i * 16).astype(jnp.float32)
        cand = jnp.where(cs >= kf, lane_g, jnp.float32(1e9))
        bmin = jnp.minimum(bmin, jnp.min(cand))
        cum = cum + jnp.sum(nzc)
        return cum, bmin

    count_nz, bminf = lax.fori_loop(
        0, NB // 16, _select, (jnp.float32(0.0), jnp.float32(1e9)))
    is_pass = count_nz <= kf
    bsafe = jnp.minimum(bminf, jnp.float32(NB - 1))

    def _cum_before(i, acc):
        nzc = histg[pl.ds(i * 16, 16)]
        lane_g = iota_f + (i * 16).astype(jnp.float32)
        return acc + jnp.sum(jnp.where(lane_g < bsafe, nzc, zero_f))

    cumb = lax.fori_loop(0, NB // 16, _cum_before, jnp.float32(0.0))
    rank_in = kf - cumb
    b_i = bsafe.astype(jnp.int32)

    off = b_i * C + sid * CW
    pltpu.sync_copy(sidx_hbm.at[pl.ds(off, CW)], idxb)
    pltpu.async_copy(x_hbm.at[idxb], valb, sema).wait()

    def _zind(i, c):
        v = valb[pl.ds(i * 16, 16)]
        zv[pl.ds(i * 16, 16)] = jnp.where(v != 0.0, one_f, zero_f)
        return c

    lax.fori_loop(0, CW // 16, _zind, 0)
    pltpu.sync_copy(zv, sh_z.at[pl.ds(sid * CW, CW)])
    plsc.subcore_barrier()

    @pl.when(sid == 0)
    def _final():
        pltpu.sync_copy(sh_z, zflat)
        pltpu.sync_copy(svali_hbm.at[pl.ds(b_i * C, C)], svalb)

        def _scan(i, carry):
            cs0, t = carry
            nz16 = zflat[pl.ds(i * 16, 16)]
            cums = plsc.cumsum(nz16) + cs0
            sv = svalb[pl.ds(i * 16, 16)]
            hit = jnp.logical_and(cums == rank_in, nz16 > 0.5)
            t = jnp.maximum(t, jnp.max(jnp.where(hit, sv, jnp.int32(-1))))
            return cs0 + jnp.sum(nz16), t

        _, t = lax.fori_loop(0, C // 16, _scan,
                             (jnp.float32(0.0), jnp.int32(-1)))
        t = jnp.where(is_pass, jnp.int32(-1), t)
        tvout[pl.ds(0, 16)] = jnp.zeros((16,), jnp.int32) + t
        pltpu.sync_copy(tvout, sh_t)

    plsc.subcore_barrier()

    pltpu.sync_copy(sh_t, tvin)
    tvec = tvin[pl.ds(0, 16)]
    mant_mask = jnp.full((16,), MANT, jnp.int32)

    def _mask(i, c):
        for u in range(8):
            o = i * 128 + u * 16
            m = metav[pl.ds(o, 16)] & mant_mask
            keep = m >= tvec
            xv[pl.ds(o, 16)] = jnp.where(keep, xv[pl.ds(o, 16)], zero_f)
        return c

    lax.fori_loop(0, PT // 128, _mask, 0)
    pltpu.sync_copy(xv, out_hbm.at[pl.ds(base, PT)])


_sc_kernel = pl.kernel(
    _sc_body,
    out_type=jax.ShapeDtypeStruct((N,), jnp.float32),
    mesh=plsc.VectorSubcoreMesh(core_axis_name="c", subcore_axis_name="s",
                                num_cores=1),
    compiler_params=pltpu.CompilerParams(needs_layout_passes=False),
    scratch_types=[
        pltpu.VMEM((PT,), jnp.float32),
        pltpu.VMEM((PT,), jnp.int32),
        pltpu.VMEM((16 * NB,), jnp.float32),
        pltpu.VMEM((NB,), jnp.float32),
        pltpu.VMEM((16 * NB,), jnp.float32),
        pltpu.VMEM((C,), jnp.float32),
        pltpu.VMEM((C,), jnp.int32),
        pltpu.VMEM((CW,), jnp.int32),
        pltpu.VMEM((CW,), jnp.float32),
        pltpu.VMEM((CW,), jnp.float32),
        pltpu.VMEM((16,), jnp.int32),
        pltpu.VMEM((16,), jnp.int32),
        pltpu.SemaphoreType.DMA,
        pltpu.SemaphoreType.DMA,
        pltpu.VMEM_SHARED((16 * NB,), jnp.float32),
        pltpu.VMEM_SHARED((C,), jnp.float32),
        pltpu.VMEM_SHARED((16,), jnp.int32),
    ],
)


def kernel(input):
    meta_c = jnp.asarray(_meta_np)
    sidx_c = jnp.asarray(_perm_np)
    svali_c = jnp.asarray(_svali_np)
    out = _sc_kernel(input.reshape(-1), meta_c, sidx_c, svali_c)
    return out.reshape(ROWS, COLS)


# speedup vs baseline: 1.3748x; 1.3748x over previous
import math

import numpy as np
import jax
import jax.numpy as jnp
from jax import lax
from jax.experimental import pallas as pl
from jax.experimental.pallas import tpu as pltpu
from jax.experimental.pallas import tpu_sc as plsc

ROWS, COLS = 64, 8192
N = ROWS * COLS
K = math.floor(0.05 * N)
C = 1024
NB = N // C
NT = 16
PT = N // NT
CW = C // NT
MANT = 0x7FFFFF


def _threefry2x32_np(k0, k1, x0, x1):
    def rotl(x, d):
        return ((x << np.uint32(d)) | (x >> np.uint32(32 - d))).astype(np.uint32)

    ks0 = np.uint32(k0)
    ks1 = np.uint32(k1)
    ks2 = np.uint32(ks0 ^ ks1 ^ np.uint32(0x1BD11BDA))
    x0 = (x0 + ks0).astype(np.uint32)
    x1 = (x1 + ks1).astype(np.uint32)
    rots = [[13, 15, 26, 6], [17, 29, 16, 24]]
    ks = [ks0, ks1, ks2]
    for i in range(5):
        for r in rots[i % 2]:
            x0 = (x0 + x1).astype(np.uint32)
            x1 = rotl(x1, r)
            x1 = (x1 ^ x0).astype(np.uint32)
        x0 = (x0 + ks[(i + 1) % 3]).astype(np.uint32)
        x1 = (x1 + ks[(i + 2) % 3] + np.uint32(i + 1)).astype(np.uint32)
    return x0, x1


def _score_mantissas_np(seed, n):
    counts = np.arange(n, dtype=np.uint64)
    o0, o1 = _threefry2x32_np(
        np.uint32(seed >> 32), np.uint32(seed & 0xFFFFFFFF),
        (counts >> np.uint64(32)).astype(np.uint32), counts.astype(np.uint32))
    return ((o0 ^ o1) >> np.uint32(9))


_u_np = _score_mantissas_np(42, N)
_perm_np = np.argsort(_u_np)[::-1].astype(np.int32)
_rank_np = np.empty(N, dtype=np.int32)
_rank_np[_perm_np] = np.arange(N, dtype=np.int32)
_seg_np = (_rank_np // C).astype(np.uint32)
_meta_np = ((_seg_np << np.uint32(23)) | _u_np).astype(np.uint32).view(np.int32)
_svali_np = _u_np[_perm_np].astype(np.uint32).view(np.int32)


def _sc_body(x_hbm, meta_hbm, sidx_hbm, svali_hbm, out_hbm,
             xv, metav, hist16, histg, histall, zflat, svalb,
             idxb, valb, zv, tvout, tvin, sema, semb, sh_hist, sh_z, sh_t):
    sid = lax.axis_index("s")
    zeros16 = jnp.zeros((16,), jnp.float32)
    iota_i = lax.iota(jnp.int32, 16)
    iota_f = iota_i.astype(jnp.float32)
    kf = jnp.float32(K)
    cf = jnp.float32(C)
    one_f = jnp.float32(1.0)
    zero_f = jnp.float32(0.0)

    NCH = 4
    PC = PT // NCH
    base = sid * PT
    cps = [(pltpu.async_copy(x_hbm.at[pl.ds(base + c * PC, PC)],
                             xv.at[pl.ds(c * PC, PC)], sema),
            pltpu.async_copy(meta_hbm.at[pl.ds(base + c * PC, PC)],
                             metav.at[pl.ds(c * PC, PC)], semb))
           for c in range(NCH)]

    @plsc.parallel_loop(0, (16 * NB) // 16, 1, unroll=8)
    def _zero_hist(i):
        hist16[pl.ds(i * 16, 16)] = zeros16

    lane_off = iota_i * NB

    for c in range(NCH):
        cps[c][0].wait()
        cps[c][1].wait()

        @plsc.parallel_loop(c * (PC // 16), (c + 1) * (PC // 16), 1, unroll=8)
        def _hist(i):
            o = i * 16
            v = xv[pl.ds(o, 16)]
            m = metav[pl.ds(o, 16)]
            seg = lax.shift_right_logical(m, 23)
            ones = jnp.where(v == 0.0, one_f, zero_f)
            plsc.addupdate_scatter(hist16, [seg + lane_off], ones)

    @plsc.parallel_loop(0, NB // 16, 1, unroll=4)
    def _lane_reduce(i):
        acc = zeros16
        for row in range(16):
            acc = acc + hist16[pl.ds(row * NB + i * 16, 16)]
        histg[pl.ds(i * 16, 16)] = acc
    pltpu.sync_copy(histg, sh_hist.at[pl.ds(sid * NB, NB)])
    plsc.subcore_barrier()

    pltpu.sync_copy(sh_hist, histall)

    @plsc.parallel_loop(0, NB // 16, 1, unroll=4)
    def _combine(i):
        acc = zeros16
        for row in range(16):
            acc = acc + histall[pl.ds(row * NB + i * 16, 16)]
        histg[pl.ds(i * 16, 16)] = cf - acc

    def _select(i, carry):
        cum, bmin = carry
        nzc = histg[pl.ds(i * 16, 16)]
        cs = plsc.cumsum(nzc) + cum
        lane_g = iota_f + (i * 16).astype(jnp.float32)
        cand = jnp.where(cs >= kf, lane_g, jnp.float32(1e9))
        bmin = jnp.minimum(bmin, jnp.min(cand))
        cum = cum + jnp.sum(nzc)
        return cum, bmin

    count_nz, bminf = lax.fori_loop(
        0, NB // 16, _select, (jnp.float32(0.0), jnp.float32(1e9)))
    is_pass = count_nz <= kf
    bsafe = jnp.minimum(bminf, jnp.float32(NB - 1))

    def _cum_before(i, acc):
        nzc = histg[pl.ds(i * 16, 16)]
        lane_g = iota_f + (i * 16).astype(jnp.float32)
        return acc + jnp.sum(jnp.where(lane_g < bsafe, nzc, zero_f))

    cumb = lax.fori_loop(0, NB // 16, _cum_before, jnp.float32(0.0))
    rank_in = kf - cumb
    b_i = bsafe.astype(jnp.int32)

    off = b_i * C + sid * CW
    pltpu.sync_copy(sidx_hbm.at[pl.ds(off, CW)], idxb)
    pltpu.async_copy(x_hbm.at[idxb], valb, sema).wait()

    def _zind(i, c):
        v = valb[pl.ds(i * 16, 16)]
        zv[pl.ds(i * 16, 16)] = jnp.where(v != 0.0, one_f, zero_f)
        return c

    lax.fori_loop(0, CW // 16, _zind, 0)
    pltpu.sync_copy(zv, sh_z.at[pl.ds(sid * CW, CW)])
    plsc.subcore_barrier()

    @pl.when(sid == 0)
    def _final():
        pltpu.sync_copy(sh_z, zflat)
        pltpu.sync_copy(svali_hbm.at[pl.ds(b_i * C, C)], svalb)

        def _scan(i, carry):
            cs0, t = carry
            nz16 = zflat[pl.ds(i * 16, 16)]
            cums = plsc.cumsum(nz16) + cs0
            sv = svalb[pl.ds(i * 16, 16)]
            hit = jnp.logical_and(cums == rank_in, nz16 > 0.5)
            t = jnp.maximum(t, jnp.max(jnp.where(hit, sv, jnp.int32(-1))))
            return cs0 + jnp.sum(nz16), t

        _, t = lax.fori_loop(0, C // 16, _scan,
                             (jnp.float32(0.0), jnp.int32(-1)))
        t = jnp.where(is_pass, jnp.int32(-1), t)
        tvout[pl.ds(0, 16)] = jnp.zeros((16,), jnp.int32) + t
        pltpu.sync_copy(tvout, sh_t)

    plsc.subcore_barrier()

    pltpu.sync_copy(sh_t, tvin)
    tvec = tvin[pl.ds(0, 16)]
    mant_mask = jnp.full((16,), MANT, jnp.int32)

    ocps = []
    for c in range(NCH):

        @plsc.parallel_loop(c * (PC // 16), (c + 1) * (PC // 16), 1, unroll=8)
        def _mask(i):
            o = i * 16
            m = metav[pl.ds(o, 16)] & mant_mask
            keep = m >= tvec
            xv[pl.ds(o, 16)] = jnp.where(keep, xv[pl.ds(o, 16)], zero_f)

        ocps.append(pltpu.async_copy(
            xv.at[pl.ds(c * PC, PC)],
            out_hbm.at[pl.ds(base + c * PC, PC)],
            sema if c % 2 == 0 else semb))
    for cp in ocps:
        cp.wait()


_sc_kernel = pl.kernel(
    _sc_body,
    out_type=jax.ShapeDtypeStruct((N,), jnp.float32),
    mesh=plsc.VectorSubcoreMesh(core_axis_name="c", subcore_axis_name="s",
                                num_cores=1),
    compiler_params=pltpu.CompilerParams(needs_layout_passes=False),
    scratch_types=[
        pltpu.VMEM((PT,), jnp.float32),
        pltpu.VMEM((PT,), jnp.int32),
        pltpu.VMEM((16 * NB,), jnp.float32),
        pltpu.VMEM((NB,), jnp.float32),
        pltpu.VMEM((16 * NB,), jnp.float32),
        pltpu.VMEM((C,), jnp.float32),
        pltpu.VMEM((C,), jnp.int32),
        pltpu.VMEM((CW,), jnp.int32),
        pltpu.VMEM((CW,), jnp.float32),
        pltpu.VMEM((CW,), jnp.float32),
        pltpu.VMEM((16,), jnp.int32),
        pltpu.VMEM((16,), jnp.int32),
        pltpu.SemaphoreType.DMA,
        pltpu.SemaphoreType.DMA,
        pltpu.VMEM_SHARED((16 * NB,), jnp.float32),
        pltpu.VMEM_SHARED((C,), jnp.float32),
        pltpu.VMEM_SHARED((16,), jnp.int32),
    ],
)


def kernel(input):
    meta_c = jnp.asarray(_meta_np)
    sidx_c = jnp.asarray(_perm_np)
    svali_c = jnp.asarray(_svali_np)
    out = _sc_kernel(input.reshape(-1), meta_c, sidx_c, svali_c)
    return out.reshape(ROWS, COLS)


# speedup vs baseline: 1.3830x; 1.0059x over previous
import math

import numpy as np
import jax
import jax.numpy as jnp
from jax import lax
from jax.experimental import pallas as pl
from jax.experimental.pallas import tpu as pltpu
from jax.experimental.pallas import tpu_sc as plsc

ROWS, COLS = 64, 8192
N = ROWS * COLS
K = math.floor(0.05 * N)
C = 1024
NB = N // C
NT = 16
PT = N // NT
CW = C // NT
MANT = 0x7FFFFF


def _threefry2x32_np(k0, k1, x0, x1):
    def rotl(x, d):
        return ((x << np.uint32(d)) | (x >> np.uint32(32 - d))).astype(np.uint32)

    ks0 = np.uint32(k0)
    ks1 = np.uint32(k1)
    ks2 = np.uint32(ks0 ^ ks1 ^ np.uint32(0x1BD11BDA))
    x0 = (x0 + ks0).astype(np.uint32)
    x1 = (x1 + ks1).astype(np.uint32)
    rots = [[13, 15, 26, 6], [17, 29, 16, 24]]
    ks = [ks0, ks1, ks2]
    for i in range(5):
        for r in rots[i % 2]:
            x0 = (x0 + x1).astype(np.uint32)
            x1 = rotl(x1, r)
            x1 = (x1 ^ x0).astype(np.uint32)
        x0 = (x0 + ks[(i + 1) % 3]).astype(np.uint32)
        x1 = (x1 + ks[(i + 2) % 3] + np.uint32(i + 1)).astype(np.uint32)
    return x0, x1


def _score_mantissas_np(seed, n):
    counts = np.arange(n, dtype=np.uint64)
    o0, o1 = _threefry2x32_np(
        np.uint32(seed >> 32), np.uint32(seed & 0xFFFFFFFF),
        (counts >> np.uint64(32)).astype(np.uint32), counts.astype(np.uint32))
    return ((o0 ^ o1) >> np.uint32(9))


_u_np = _score_mantissas_np(42, N)
_perm_np = np.argsort(_u_np)[::-1].astype(np.int32)
_rank_np = np.empty(N, dtype=np.int32)
_rank_np[_perm_np] = np.arange(N, dtype=np.int32)
_seg_np = (_rank_np // C).astype(np.uint64)
_svali_np = _u_np[_perm_np]
_ends_np = np.flatnonzero(np.append(_svali_np[1:] != _svali_np[:-1], True))
_tiehi_np = _ends_np[np.searchsorted(_ends_np, np.arange(N))].astype(np.int32)
_scat_np = (np.arange(N, dtype=np.uint64) % 16) * NB + _seg_np
_word_np = ((_scat_np << np.uint64(19))
            | _rank_np.astype(np.uint64)).astype(np.uint32).view(np.int32)


def _sc_body(x_hbm, word_hbm, sidx_hbm, tiehi_hbm, out_hbm,
             xv, wv, hist16, histg, histall, zflat,
             idxb, valb, zv, jv, tout, tvout, tvin, sema, semb,
             sh_hist, sh_z, sh_t):
    sid = lax.axis_index("s")
    zeros16 = jnp.zeros((16,), jnp.float32)
    iota_i = lax.iota(jnp.int32, 16)
    iota_f = iota_i.astype(jnp.float32)
    kf = jnp.float32(K)
    cf = jnp.float32(C)
    one_f = jnp.float32(1.0)
    zero_f = jnp.float32(0.0)

    NCH = 4
    PC = PT // NCH
    base = sid * PT
    cps = [(pltpu.async_copy(x_hbm.at[pl.ds(base + c * PC, PC)],
                             xv.at[pl.ds(c * PC, PC)], sema),
            pltpu.async_copy(word_hbm.at[pl.ds(base + c * PC, PC)],
                             wv.at[pl.ds(c * PC, PC)], semb))
           for c in range(NCH)]

    @plsc.parallel_loop(0, (16 * NB) // 16, 1, unroll=8)
    def _zero_hist(i):
        hist16[pl.ds(i * 16, 16)] = zeros16

    for c in range(NCH):
        cps[c][0].wait()
        cps[c][1].wait()

        @plsc.parallel_loop(c * (PC // 16), (c + 1) * (PC // 16), 1, unroll=8)
        def _hist(i):
            o = i * 16
            v = xv[pl.ds(o, 16)]
            w = wv[pl.ds(o, 16)]
            idx = lax.shift_right_logical(w, 19)
            ones = jnp.where(v == 0.0, one_f, zero_f)
            plsc.addupdate_scatter(hist16, [idx], ones)

    @plsc.parallel_loop(0, NB // 16, 1, unroll=4)
    def _lane_reduce(i):
        acc = zeros16
        for row in range(16):
            acc = acc + hist16[pl.ds(row * NB + i * 16, 16)]
        histg[pl.ds(i * 16, 16)] = acc
    pltpu.sync_copy(histg, sh_hist.at[pl.ds(sid * NB, NB)])
    plsc.subcore_barrier()

    pltpu.sync_copy(sh_hist, histall)

    @plsc.parallel_loop(0, NB // 16, 1, unroll=4)
    def _combine(i):
        acc = zeros16
        for row in range(16):
            acc = acc + histall[pl.ds(row * NB + i * 16, 16)]
        histg[pl.ds(i * 16, 16)] = cf - acc

    def _select(i, carry):
        cum, bmin = carry
        nzc = histg[pl.ds(i * 16, 16)]
        cs = plsc.cumsum(nzc) + cum
        lane_g = iota_f + (i * 16).astype(jnp.float32)
        cand = jnp.where(cs >= kf, lane_g, jnp.float32(1e9))
        bmin = jnp.minimum(bmin, jnp.min(cand))
        cum = cum + jnp.sum(nzc)
        return cum, bmin

    count_nz, bminf = lax.fori_loop(
        0, NB // 16, _select, (jnp.float32(0.0), jnp.float32(1e9)))
    is_pass = count_nz <= kf
    bsafe = jnp.minimum(bminf, jnp.float32(NB - 1))

    def _cum_before(i, acc):
        nzc = histg[pl.ds(i * 16, 16)]
        lane_g = iota_f + (i * 16).astype(jnp.float32)
        return acc + jnp.sum(jnp.where(lane_g < bsafe, nzc, zero_f))

    cumb = lax.fori_loop(0, NB // 16, _cum_before, jnp.float32(0.0))
    rank_in = kf - cumb
    b_i = bsafe.astype(jnp.int32)

    off = b_i * C + sid * CW
    pltpu.sync_copy(sidx_hbm.at[pl.ds(off, CW)], idxb)
    pltpu.async_copy(x_hbm.at[idxb], valb, sema).wait()

    def _zind(i, c):
        v = valb[pl.ds(i * 16, 16)]
        zv[pl.ds(i * 16, 16)] = jnp.where(v != 0.0, one_f, zero_f)
        return c

    lax.fori_loop(0, CW // 16, _zind, 0)
    pltpu.sync_copy(zv, sh_z.at[pl.ds(sid * CW, CW)])
    plsc.subcore_barrier()

    @pl.when(sid == 0)
    def _final():
        pltpu.sync_copy(sh_z, zflat)

        def _scan(i, carry):
            cs0, js = carry
            nz16 = zflat[pl.ds(i * 16, 16)]
            cums = plsc.cumsum(nz16) + cs0
            hit = jnp.logical_and(cums == rank_in, nz16 > 0.5)
            pos = iota_i + (b_i * C + i * 16)
            js = jnp.maximum(js, jnp.max(jnp.where(hit, pos, jnp.int32(-1))))
            return cs0 + jnp.sum(nz16), js

        _, jstar = lax.fori_loop(0, C // 16, _scan,
                                 (jnp.float32(0.0), jnp.int32(-1)))
        jstar = jnp.where(is_pass, jnp.int32(0), jstar)
        jv[pl.ds(0, 16)] = jnp.zeros((16,), jnp.int32) + jstar
        pltpu.async_copy(tiehi_hbm.at[jv], tout, sema).wait()
        th = tout[pl.ds(0, 16)]
        tvout[pl.ds(0, 16)] = jnp.where(is_pass, jnp.int32(N - 1), th)
        pltpu.sync_copy(tvout, sh_t)

    plsc.subcore_barrier()

    pltpu.sync_copy(sh_t, tvin)
    jhivec = tvin[pl.ds(0, 16)]
    rank_mask = jnp.full((16,), (1 << 19) - 1, jnp.int32)

    ocps = []
    for c in range(NCH):

        @plsc.parallel_loop(c * (PC // 16), (c + 1) * (PC // 16), 1, unroll=8)
        def _mask(i):
            o = i * 16
            rk = wv[pl.ds(o, 16)] & rank_mask
            keep = rk <= jhivec
            xv[pl.ds(o, 16)] = jnp.where(keep, xv[pl.ds(o, 16)], zero_f)

        ocps.append(pltpu.async_copy(
            xv.at[pl.ds(c * PC, PC)],
            out_hbm.at[pl.ds(base + c * PC, PC)],
            sema if c % 2 == 0 else semb))
    for cp in ocps:
        cp.wait()


_sc_kernel = pl.kernel(
    _sc_body,
    out_type=jax.ShapeDtypeStruct((N,), jnp.float32),
    mesh=plsc.VectorSubcoreMesh(core_axis_name="c", subcore_axis_name="s",
                                num_cores=1),
    compiler_params=pltpu.CompilerParams(needs_layout_passes=False),
    scratch_types=[
        pltpu.VMEM((PT,), jnp.float32),
        pltpu.VMEM((PT,), jnp.int32),
        pltpu.VMEM((16 * NB,), jnp.float32),
        pltpu.VMEM((NB,), jnp.float32),
        pltpu.VMEM((16 * NB,), jnp.float32),
        pltpu.VMEM((C,), jnp.float32),
        pltpu.VMEM((CW,), jnp.int32),
        pltpu.VMEM((CW,), jnp.float32),
        pltpu.VMEM((CW,), jnp.float32),
        pltpu.VMEM((16,), jnp.int32),
        pltpu.VMEM((16,), jnp.int32),
        pltpu.VMEM((16,), jnp.int32),
        pltpu.VMEM((16,), jnp.int32),
        pltpu.SemaphoreType.DMA,
        pltpu.SemaphoreType.DMA,
        pltpu.VMEM_SHARED((16 * NB,), jnp.float32),
        pltpu.VMEM_SHARED((C,), jnp.float32),
        pltpu.VMEM_SHARED((16,), jnp.int32),
    ],
)


def kernel(input):
    word_c = jnp.asarray(_word_np)
    sidx_c = jnp.asarray(_perm_np)
    tiehi_c = jnp.asarray(_tiehi_np)
    out = _sc_kernel(input.reshape(-1), word_c, sidx_c, tiehi_c)
    return out.reshape(ROWS, COLS)
